# baseline (speedup 1.0000x reference)
import functools

import jax
import jax.numpy as jnp
from jax import lax
from jax.experimental import pallas as pl
from jax.experimental.pallas import tpu as pltpu
from jax.experimental.pallas import tpu_sc as plsc

N = 10000
E = 320000
D = 128

NC = 2
NS = 16
NW = NC * NS

CHUNK = 80
NSLOT = 4
NGROUP = 32
NCHUNK = NSLOT * NGROUP
EPW = NCHUNK * CHUNK
EPAD = NW * EPW

NPAD = 10240
RPT = NPAD // NS


def _sc_aggregate(x, src, dst):
    mesh = plsc.VectorSubcoreMesh(core_axis_name="c", subcore_axis_name="s")

    @functools.partial(
        pl.kernel,
        mesh=mesh,
        out_type=jax.ShapeDtypeStruct((NC, NPAD, D), jnp.float32),
        scratch_types=(
            [pltpu.VMEM((CHUNK,), jnp.int32) for _ in range(NSLOT)]
            + [pltpu.VMEM((CHUNK,), jnp.int32) for _ in range(NSLOT)]
            + [pltpu.VMEM((CHUNK, D), jnp.float32) for _ in range(NSLOT)]
            + [pltpu.VMEM_SHARED((NPAD, D), jnp.float32)]
            + [pltpu.SemaphoreType.DMA for _ in range(3 * NSLOT)]
        ),
    )
    def agg_kernel(x_hbm, src_hbm, dst_hbm, out_hbm, *scr):
        sidx = scr[0:NSLOT]
        didx = scr[NSLOT:2 * NSLOT]
        rows = scr[2 * NSLOT:3 * NSLOT]
        acc = scr[3 * NSLOT]
        isem = scr[3 * NSLOT + 1:3 * NSLOT + 1 + NSLOT]
        gsem = scr[3 * NSLOT + 1 + NSLOT:3 * NSLOT + 1 + 2 * NSLOT]
        ssem = scr[3 * NSLOT + 1 + 2 * NSLOT:3 * NSLOT + 1 + 3 * NSLOT]

        c = lax.axis_index("c")
        s = lax.axis_index("s")
        wid = s * NC + c

        def zero_row(r, carry):
            for c0 in range(0, D, 16):
                rows[0][r, pl.ds(c0, 16)] = jnp.zeros((16,), jnp.float32)
            return carry
        lax.fori_loop(0, CHUNK, zero_row, 0)
        for t in range(RPT // CHUNK):
            pltpu.sync_copy(rows[0], acc.at[pl.ds(s * RPT + t * CHUNK, CHUNK)])
        plsc.subcore_barrier()

        ebase = wid * EPW

        def body(j, carry):
            off = ebase + j * CHUNK
            pltpu.sync_copy(src_hbm.at[pl.ds(off, CHUNK)], sidx[0])
            pltpu.sync_copy(dst_hbm.at[pl.ds(off, CHUNK)], didx[0])
            pltpu.sync_copy(x_hbm.at[sidx[0]], rows[0])
            pltpu.sync_copy(rows[0], acc.at[didx[0]], add=True)
            return carry
        lax.fori_loop(0, NCHUNK, body, 0)
        plsc.subcore_barrier()

        wdescs = []
        for t in range(RPT // CHUNK):
            r0 = s * RPT + t * CHUNK
            wdescs.append(
                pltpu.async_copy(acc.at[pl.ds(r0, CHUNK)],
                                 out_hbm.at[c, pl.ds(r0, CHUNK)],
                                 gsem[t % NSLOT]))
        for d in wdescs:
            d.wait()

    return agg_kernel(x, src, dst)


def _mlp_kernel(x_ref, p_ref, w1_ref, b1_ref, g_ref, be_ref, w2_ref, b2_ref,
                o_ref):
    h = x_ref[...] + p_ref[0, :N, :] + p_ref[1, :N, :]
    z = lax.dot_general(h, w1_ref[...], (((1,), (1,)), ((), ())),
                        preferred_element_type=jnp.float32)
    z = jnp.maximum(z + b1_ref[...], 0.0)
    mean = jnp.mean(z, axis=0, keepdims=True)
    var = jnp.mean(z * z, axis=0, keepdims=True) - mean * mean
    scale = g_ref[...] * lax.rsqrt(var + 1e-5)
    zn = (z - mean) * scale + be_ref[...]
    o_ref[...] = lax.dot_general(zn, w2_ref[...], (((1,), (1,)), ((), ())),
                                 preferred_element_type=jnp.float32) + b2_ref[...]


def _mlp(x, parts, W1, b1, gamma, beta, W2, b2):
    return pl.pallas_call(
        _mlp_kernel,
        out_shape=jax.ShapeDtypeStruct((N, D), jnp.float32),
    )(x, parts, W1, b1.reshape(1, D), gamma.reshape(1, D),
      beta.reshape(1, D), W2, b2.reshape(1, D))


def kernel(x, edge_index, W1, b1, gamma, beta, W2, b2):
    npad_e = EPAD - E
    src = jnp.concatenate([edge_index[0], jnp.zeros((npad_e,), jnp.int32)])
    pad_dst = N + (jnp.arange(npad_e, dtype=jnp.int32) % (NPAD - N))
    dst = jnp.concatenate([edge_index[1], pad_dst])
    parts = _sc_aggregate(x, src, dst)
    return _mlp(x, parts, W1, b1, gamma, beta, W2, b2)


# speedup vs baseline: 3.5641x; 3.5641x over previous
import functools

import jax
import jax.numpy as jnp
from jax import lax
from jax.experimental import pallas as pl
from jax.experimental.pallas import tpu as pltpu
from jax.experimental.pallas import tpu_sc as plsc

N = 10000
E = 320000
D = 128

NC = 2
NS = 16
NW = NC * NS

CHUNK = 80
NSLOT = 4
EPW = E // NW
NCHUNK = EPW // CHUNK
NGROUP = NCHUNK // NSLOT

NPAD = 10240
RPT = NPAD // NS


def _sc_aggregate(x, edge_index):
    mesh = plsc.VectorSubcoreMesh(core_axis_name="c", subcore_axis_name="s")

    @functools.partial(
        pl.kernel,
        mesh=mesh,
        out_type=jax.ShapeDtypeStruct((NC, NPAD, D), jnp.float32),
        scratch_types=(
            [pltpu.VMEM((CHUNK,), jnp.int32) for _ in range(NSLOT)]
            + [pltpu.VMEM((CHUNK,), jnp.int32) for _ in range(NSLOT)]
            + [pltpu.VMEM((CHUNK, D), jnp.float32) for _ in range(NSLOT)]
            + [pltpu.VMEM_SHARED((NPAD, D), jnp.float32)]
            + [pltpu.SemaphoreType.DMA for _ in range(3 * NSLOT)]
        ),
    )
    def agg_kernel(x_hbm, src_hbm, dst_hbm, out_hbm, *scr):
        sidx = scr[0:NSLOT]
        didx = scr[NSLOT:2 * NSLOT]
        rows = scr[2 * NSLOT:3 * NSLOT]
        acc = scr[3 * NSLOT]
        isem = scr[3 * NSLOT + 1:3 * NSLOT + 1 + NSLOT]
        gsem = scr[3 * NSLOT + 1 + NSLOT:3 * NSLOT + 1 + 2 * NSLOT]
        ssem = scr[3 * NSLOT + 1 + 2 * NSLOT:3 * NSLOT + 1 + 3 * NSLOT]

        c = lax.axis_index("c")
        s = lax.axis_index("s")
        wid = s * NC + c

        def zero_row(r, carry):
            for c0 in range(0, D, 16):
                rows[0][r, pl.ds(c0, 16)] = jnp.zeros((16,), jnp.float32)
            return carry
        lax.fori_loop(0, CHUNK, zero_row, 0)
        for t in range(RPT // CHUNK):
            pltpu.sync_copy(rows[0], acc.at[pl.ds(s * RPT + t * CHUNK, CHUNK)])
        plsc.subcore_barrier()

        ebase = wid * EPW

        def group(k, carry):
            @pl.when(k > 0)
            def _():
                for t in range(NSLOT):
                    pltpu.make_async_copy(
                        rows[t], acc.at[didx[t]], ssem[t]).wait()
            idescs = []
            for t in range(NSLOT):
                off = ebase + (k * NSLOT + t) * CHUNK
                idescs.append((
                    pltpu.async_copy(src_hbm.at[pl.ds(off, CHUNK)],
                                     sidx[t], isem[t]),
                    pltpu.async_copy(dst_hbm.at[pl.ds(off, CHUNK)],
                                     didx[t], isem[t]),
                ))
            gdescs = []
            for t in range(NSLOT):
                idescs[t][0].wait()
                idescs[t][1].wait()
                gdescs.append(
                    pltpu.async_copy(x_hbm.at[sidx[t]], rows[t], gsem[t]))
            for t in range(NSLOT):
                gdescs[t].wait()
                pltpu.async_copy(rows[t], acc.at[didx[t]], ssem[t], add=True)
            return carry
        lax.fori_loop(0, NGROUP, group, 0)
        for t in range(NSLOT):
            pltpu.make_async_copy(rows[t], acc.at[didx[t]], ssem[t]).wait()
        off = ebase + NGROUP * NSLOT * CHUNK
        pltpu.sync_copy(src_hbm.at[pl.ds(off, CHUNK)], sidx[0])
        pltpu.sync_copy(dst_hbm.at[pl.ds(off, CHUNK)], didx[0])
        pltpu.sync_copy(x_hbm.at[sidx[0]], rows[0])
        pltpu.sync_copy(rows[0], acc.at[didx[0]], add=True)
        plsc.subcore_barrier()

        wdescs = []
        for t in range(RPT // CHUNK):
            r0 = s * RPT + t * CHUNK
            wdescs.append(
                pltpu.async_copy(acc.at[pl.ds(r0, CHUNK)],
                                 out_hbm.at[c, pl.ds(r0, CHUNK)],
                                 gsem[t % NSLOT]))
        for d in wdescs:
            d.wait()

    return agg_kernel(x, edge_index[0], edge_index[1])


def _mlp_kernel(x_ref, p_ref, w1_ref, b1_ref, g_ref, be_ref, w2_ref, b2_ref,
                o_ref):
    h = x_ref[...] + p_ref[0, :N, :] + p_ref[1, :N, :]
    z = lax.dot_general(h, w1_ref[...], (((1,), (1,)), ((), ())),
                        preferred_element_type=jnp.float32)
    z = jnp.maximum(z + b1_ref[...], 0.0)
    mean = jnp.mean(z, axis=0, keepdims=True)
    var = jnp.mean(z * z, axis=0, keepdims=True) - mean * mean
    scale = g_ref[...] * lax.rsqrt(var + 1e-5)
    zn = (z - mean) * scale + be_ref[...]
    o_ref[...] = lax.dot_general(zn, w2_ref[...], (((1,), (1,)), ((), ())),
                                 preferred_element_type=jnp.float32) + b2_ref[...]


def _mlp(x, parts, W1, b1, gamma, beta, W2, b2):
    return pl.pallas_call(
        _mlp_kernel,
        out_shape=jax.ShapeDtypeStruct((N, D), jnp.float32),
    )(x, parts, W1, b1.reshape(1, D), gamma.reshape(1, D),
      beta.reshape(1, D), W2, b2.reshape(1, D))


def kernel(x, edge_index, W1, b1, gamma, beta, W2, b2):
    parts = _sc_aggregate(x, edge_index)
    return _mlp(x, parts, W1, b1, gamma, beta, W2, b2)


# speedup vs baseline: 3.8221x; 1.0724x over previous
import functools

import jax
import jax.numpy as jnp
from jax import lax
from jax.experimental import pallas as pl
from jax.experimental.pallas import tpu as pltpu
from jax.experimental.pallas import tpu_sc as plsc

N = 10000
E = 320000
D = 128

NC = 2
NS = 16
NW = NC * NS

CHUNK = 80
NSLOT = 4
EPW = E // NW
NCHUNK = EPW // CHUNK
NGROUP = NCHUNK // NSLOT

NPAD = 10240
RPT = NPAD // NS


def _sc_aggregate(x, edge_index):
    mesh = plsc.VectorSubcoreMesh(core_axis_name="c", subcore_axis_name="s")

    @functools.partial(
        pl.kernel,
        mesh=mesh,
        out_type=jax.ShapeDtypeStruct((NC, NPAD, D), jnp.float32),
        scratch_types=(
            [pltpu.VMEM((CHUNK,), jnp.int32) for _ in range(2 * NSLOT)]
            + [pltpu.VMEM((CHUNK,), jnp.int32) for _ in range(2 * NSLOT)]
            + [pltpu.VMEM((CHUNK, D), jnp.float32) for _ in range(NSLOT)]
            + [pltpu.VMEM_SHARED((NPAD, D), jnp.float32)]
            + [pltpu.SemaphoreType.DMA for _ in range(4 * NSLOT)]
        ),
    )
    def agg_kernel(x_hbm, src_hbm, dst_hbm, out_hbm, *scr):
        sidx = (scr[0:NSLOT], scr[NSLOT:2 * NSLOT])
        didx = (scr[2 * NSLOT:3 * NSLOT], scr[3 * NSLOT:4 * NSLOT])
        rows = scr[4 * NSLOT:5 * NSLOT]
        acc = scr[5 * NSLOT]
        base = 5 * NSLOT + 1
        isem = (scr[base:base + NSLOT], scr[base + NSLOT:base + 2 * NSLOT])
        gsem = scr[base + 2 * NSLOT:base + 3 * NSLOT]
        ssem = scr[base + 3 * NSLOT:base + 4 * NSLOT]

        c = lax.axis_index("c")
        s = lax.axis_index("s")
        wid = s * NC + c
        ebase = wid * EPW

        def issue_idx(k, p, t):
            off = ebase + (k * NSLOT + t) * CHUNK
            pltpu.async_copy(src_hbm.at[pl.ds(off, CHUNK)],
                             sidx[p][t], isem[p][t])
            pltpu.async_copy(dst_hbm.at[pl.ds(off, CHUNK)],
                             didx[p][t], isem[p][t])

        for t in range(NSLOT):
            issue_idx(0, 0, t)

        def zero_row(r, carry):
            for c0 in range(0, D, 16):
                rows[0][r, pl.ds(c0, 16)] = jnp.zeros((16,), jnp.float32)
            return carry
        lax.fori_loop(0, CHUNK, zero_row, 0)
        for t in range(RPT // CHUNK):
            pltpu.sync_copy(rows[0], acc.at[pl.ds(s * RPT + t * CHUNK, CHUNK)])
        plsc.subcore_barrier()

        def wait_idx(k, p, t):
            off = ebase + (k * NSLOT + t) * CHUNK
            pltpu.make_async_copy(src_hbm.at[pl.ds(off, CHUNK)],
                                  sidx[p][t], isem[p][t]).wait()
            pltpu.make_async_copy(dst_hbm.at[pl.ds(off, CHUNK)],
                                  didx[p][t], isem[p][t]).wait()

        def wait_scatter(t):
            pltpu.make_async_copy(rows[t], acc.at[didx[0][t]], ssem[t]).wait()

        def run_group(k, cur, nxt, prefetch, first):
            if first:
                @pl.when(k > 0)
                def _():
                    for t in range(NSLOT):
                        wait_scatter(t)
            else:
                for t in range(NSLOT):
                    wait_scatter(t)
            if prefetch:
                for t in range(NSLOT):
                    issue_idx(k + 1, nxt, t)
            gdescs = []
            for t in range(NSLOT):
                wait_idx(k, cur, t)
                gdescs.append(
                    pltpu.async_copy(x_hbm.at[sidx[cur][t]], rows[t],
                                     gsem[t]))
            for t in range(NSLOT):
                gdescs[t].wait()
                pltpu.async_copy(rows[t], acc.at[didx[cur][t]], ssem[t],
                                 add=True)

        def dgroup(m, carry):
            k = 2 * m
            run_group(k, 0, 1, True, True)
            run_group(k + 1, 1, 0, True, False)
            return carry
        lax.fori_loop(0, (NGROUP - 1) // 2, dgroup, 0)
        run_group(NGROUP - 1, 0, 1, False, False)
        for t in range(NSLOT):
            wait_scatter(t)
        off = ebase + NGROUP * NSLOT * CHUNK
        pltpu.sync_copy(src_hbm.at[pl.ds(off, CHUNK)], sidx[0][0])
        pltpu.sync_copy(dst_hbm.at[pl.ds(off, CHUNK)], didx[0][0])
        pltpu.sync_copy(x_hbm.at[sidx[0][0]], rows[0])
        pltpu.sync_copy(rows[0], acc.at[didx[0][0]], add=True)
        plsc.subcore_barrier()

        wdescs = []
        for t in range(RPT // CHUNK):
            r0 = s * RPT + t * CHUNK
            wdescs.append(
                pltpu.async_copy(acc.at[pl.ds(r0, CHUNK)],
                                 out_hbm.at[c, pl.ds(r0, CHUNK)],
                                 gsem[t % NSLOT]))
        for d in wdescs:
            d.wait()

    return agg_kernel(x, edge_index[0], edge_index[1])


def _mlp_kernel(x_ref, p_ref, w1_ref, b1_ref, g_ref, be_ref, w2_ref, b2_ref,
                o_ref):
    h = x_ref[...] + p_ref[0, :N, :] + p_ref[1, :N, :]
    z = lax.dot_general(h, w1_ref[...], (((1,), (1,)), ((), ())),
                        preferred_element_type=jnp.float32)
    z = jnp.maximum(z + b1_ref[...], 0.0)
    mean = jnp.mean(z, axis=0, keepdims=True)
    var = jnp.mean(z * z, axis=0, keepdims=True) - mean * mean
    scale = g_ref[...] * lax.rsqrt(var + 1e-5)
    zn = (z - mean) * scale + be_ref[...]
    o_ref[...] = lax.dot_general(zn, w2_ref[...], (((1,), (1,)), ((), ())),
                                 preferred_element_type=jnp.float32) + b2_ref[...]


def _mlp(x, parts, W1, b1, gamma, beta, W2, b2):
    return pl.pallas_call(
        _mlp_kernel,
        out_shape=jax.ShapeDtypeStruct((N, D), jnp.float32),
    )(x, parts, W1, b1.reshape(1, D), gamma.reshape(1, D),
      beta.reshape(1, D), W2, b2.reshape(1, D))


def kernel(x, edge_index, W1, b1, gamma, beta, W2, b2):
    parts = _sc_aggregate(x, edge_index)
    return _mlp(x, parts, W1, b1, gamma, beta, W2, b2)


# speedup vs baseline: 4.0469x; 1.0588x over previous
import functools

import jax
import jax.numpy as jnp
from jax import lax
from jax.experimental import pallas as pl
from jax.experimental.pallas import tpu as pltpu
from jax.experimental.pallas import tpu_sc as plsc

N = 10000
E = 320000
D = 128

NC = 2
NS = 16
NW = NC * NS

CHUNK = 80
NSLOT = 4
EPW = E // NW
NCHUNK = EPW // CHUNK
NGROUP = NCHUNK // NSLOT

NPAD = 10240
RPT = NPAD // NS


def _sc_aggregate(x, edge_index):
    mesh = plsc.VectorSubcoreMesh(core_axis_name="c", subcore_axis_name="s")

    @functools.partial(
        pl.kernel,
        mesh=mesh,
        out_type=jax.ShapeDtypeStruct((NC, NPAD, D), jnp.float32),
        scratch_types=(
            [pltpu.VMEM((CHUNK,), jnp.int32) for _ in range(2 * NSLOT)]
            + [pltpu.VMEM((CHUNK,), jnp.int32) for _ in range(2 * NSLOT)]
            + [pltpu.VMEM((CHUNK, D), jnp.float32) for _ in range(NSLOT)]
            + [pltpu.VMEM_SHARED((NPAD, D), jnp.float32)]
            + [pltpu.SemaphoreType.DMA for _ in range(4 * NSLOT)]
        ),
    )
    def agg_kernel(x_hbm, e_hbm, out_hbm, *scr):
        sidx = (scr[0:NSLOT], scr[NSLOT:2 * NSLOT])
        didx = (scr[2 * NSLOT:3 * NSLOT], scr[3 * NSLOT:4 * NSLOT])
        rows = scr[4 * NSLOT:5 * NSLOT]
        acc = scr[5 * NSLOT]
        base = 5 * NSLOT + 1
        isem = (scr[base:base + NSLOT], scr[base + NSLOT:base + 2 * NSLOT])
        gsem = scr[base + 2 * NSLOT:base + 3 * NSLOT]
        ssem = scr[base + 3 * NSLOT:base + 4 * NSLOT]

        c = lax.axis_index("c")
        s = lax.axis_index("s")
        wid = s * NC + c
        ebase = wid * EPW

        def issue_idx(k, p, t):
            off = ebase + (k * NSLOT + t) * CHUNK
            pltpu.async_copy(e_hbm.at[pl.ds(off, CHUNK)],
                             sidx[p][t], isem[p][t])
            pltpu.async_copy(e_hbm.at[pl.ds(E + off, CHUNK)],
                             didx[p][t], isem[p][t])

        for t in range(NSLOT):
            issue_idx(0, 0, t)

        def zero_row(r, carry):
            for c0 in range(0, D, 16):
                rows[0][r, pl.ds(c0, 16)] = jnp.zeros((16,), jnp.float32)
            return carry
        lax.fori_loop(0, CHUNK, zero_row, 0)
        for t in range(RPT // CHUNK):
            pltpu.sync_copy(rows[0], acc.at[pl.ds(s * RPT + t * CHUNK, CHUNK)])
        plsc.subcore_barrier()

        def wait_idx(k, p, t):
            off = ebase + (k * NSLOT + t) * CHUNK
            pltpu.make_async_copy(e_hbm.at[pl.ds(off, CHUNK)],
                                  sidx[p][t], isem[p][t]).wait()
            pltpu.make_async_copy(e_hbm.at[pl.ds(E + off, CHUNK)],
                                  didx[p][t], isem[p][t]).wait()

        def wait_scatter(t):
            pltpu.make_async_copy(rows[t], acc.at[didx[0][t]], ssem[t]).wait()

        def run_group(k, cur, nxt, prefetch, first):
            if first:
                @pl.when(k > 0)
                def _():
                    for t in range(NSLOT):
                        wait_scatter(t)
            else:
                for t in range(NSLOT):
                    wait_scatter(t)
            if prefetch:
                for t in range(NSLOT):
                    issue_idx(k + 1, nxt, t)
            gdescs = []
            for t in range(NSLOT):
                wait_idx(k, cur, t)
                gdescs.append(
                    pltpu.async_copy(x_hbm.at[sidx[cur][t]], rows[t],
                                     gsem[t]))
            for t in range(NSLOT):
                gdescs[t].wait()
                pltpu.async_copy(rows[t], acc.at[didx[cur][t]], ssem[t],
                                 add=True)

        def dgroup(m, carry):
            k = 2 * m
            run_group(k, 0, 1, True, True)
            run_group(k + 1, 1, 0, True, False)
            return carry
        lax.fori_loop(0, (NGROUP - 1) // 2, dgroup, 0)
        run_group(NGROUP - 1, 0, 1, False, False)
        for t in range(NSLOT):
            wait_scatter(t)
        off = ebase + NGROUP * NSLOT * CHUNK
        pltpu.sync_copy(e_hbm.at[pl.ds(off, CHUNK)], sidx[0][0])
        pltpu.sync_copy(e_hbm.at[pl.ds(E + off, CHUNK)], didx[0][0])
        pltpu.sync_copy(x_hbm.at[sidx[0][0]], rows[0])
        pltpu.sync_copy(rows[0], acc.at[didx[0][0]], add=True)
        plsc.subcore_barrier()

        wdescs = []
        for t in range(RPT // CHUNK):
            r0 = s * RPT + t * CHUNK
            wdescs.append(
                pltpu.async_copy(acc.at[pl.ds(r0, CHUNK)],
                                 out_hbm.at[c, pl.ds(r0, CHUNK)],
                                 gsem[t % NSLOT]))
        for d in wdescs:
            d.wait()

    return agg_kernel(x, edge_index.reshape(2 * E))


def _mlp_kernel(x_ref, p_ref, w1_ref, b1_ref, g_ref, be_ref, w2_ref, b2_ref,
                o_ref):
    h = x_ref[...] + p_ref[0, :N, :] + p_ref[1, :N, :]
    z = lax.dot_general(h, w1_ref[...], (((1,), (1,)), ((), ())),
                        preferred_element_type=jnp.float32)
    z = jnp.maximum(z + b1_ref[...], 0.0)
    mean = jnp.mean(z, axis=0, keepdims=True)
    var = jnp.mean(z * z, axis=0, keepdims=True) - mean * mean
    scale = g_ref[...] * lax.rsqrt(var + 1e-5)
    zn = (z - mean) * scale + be_ref[...]
    o_ref[...] = lax.dot_general(zn, w2_ref[...], (((1,), (1,)), ((), ())),
                                 preferred_element_type=jnp.float32) + b2_ref[...]


def _mlp(x, parts, W1, b1, gamma, beta, W2, b2):
    return pl.pallas_call(
        _mlp_kernel,
        out_shape=jax.ShapeDtypeStruct((N, D), jnp.float32),
    )(x, parts, W1, b1.reshape(1, D), gamma.reshape(1, D),
      beta.reshape(1, D), W2, b2.reshape(1, D))


def kernel(x, edge_index, W1, b1, gamma, beta, W2, b2):
    parts = _sc_aggregate(x, edge_index)
    return _mlp(x, parts, W1, b1, gamma, beta, W2, b2)
